# Initial kernel scaffold; baseline (speedup 1.0000x reference)
import jax
import jax.numpy as jnp
from jax.experimental import pallas as pl


def kernel(pred, labels, offsets, landmarks):
    raise NotImplementedError("write your pallas kernel here")


# speedup vs baseline: 1.3950x; 1.3950x over previous
import functools

import jax
import jax.numpy as jnp
from jax import lax
from jax.experimental import pallas as pl
from jax.experimental.pallas import tpu as pltpu
from jax.experimental.pallas import tpu_sc as plsc

N = 1048576
NC = 2
NS = 16
NW = NC * NS
L = 16
RW = N // NW

RTC = 2048
CH1 = 2048
CH2 = 1024

B1 = 512
B2 = 1024

CLS_W = 1.0
BBOX_W = 0.5
LMK_W = 0.5

_G_COEF = (1.8498544538905285e-04, 2.8751506391739456e-04,
           -5.4268610571399910e-03, 8.3107776364009530e-05,
           1.2498464620813230e-01, -4.9999884358222030e-01,
           6.9314715967354310e-01)

_MESH = plsc.VectorSubcoreMesh(core_axis_name="c", subcore_axis_name="s")
_CPARAMS = pltpu.CompilerParams(needs_layout_passes=False)


def _g_poly(s):
    acc = jnp.full(s.shape, _G_COEF[0], jnp.float32)
    for c in _G_COEF[1:]:
        acc = acc * s + c
    return acc


def _tc_body(pred_ref, lab_ref, off_ref, lmk_ref, vc_ref, vo_ref, vl_ref):
    p = pred_ref[...]
    lbl = lab_ref[...]
    off = off_ref[...]
    lmk = lmk_ref[...]

    z = p[:, 0]
    s = 1.0 / (1.0 + jnp.exp(-z))
    y = jnp.where(lbl == 1, 1.0, 0.0)
    per_cls = s * (1.0 - y) + _g_poly(s)
    keep = lbl >= 0

    d = p[:, 1:5] - off
    per_off = jnp.sum(d * d, axis=1) * 0.25
    offm = (lbl == 1) | (lbl == -1)

    e = p[:, 5:15] - lmk
    per_lmk = jnp.sum(e * e, axis=1) * 0.1
    lmkm = lbl == -2

    vc_ref[...] = jnp.where(keep, per_cls, -1.0)
    vo_ref[...] = jnp.where(offm, per_off, -1.0)
    vl_ref[...] = jnp.where(lmkm, per_lmk, -1.0)


_tc_values = pl.pallas_call(
    _tc_body,
    grid=(N // RTC,),
    in_specs=[
        pl.BlockSpec((RTC, 15), lambda i: (i, 0)),
        pl.BlockSpec((RTC,), lambda i: (i,)),
        pl.BlockSpec((RTC, 4), lambda i: (i, 0)),
        pl.BlockSpec((RTC, 10), lambda i: (i, 0)),
    ],
    out_specs=[pl.BlockSpec((RTC,), lambda i: (i,))] * 3,
    out_shape=[jax.ShapeDtypeStruct((N,), jnp.float32)] * 3,
)


def _wid():
    return lax.axis_index("s") * NC + lax.axis_index("c")


def _zero_ref(ref, nwords):
    z = jnp.zeros((L,), jnp.float32)

    @pl.loop(0, nwords // L)
    def _(i):
        ref[pl.ds(i * L, L)] = z


def _lane_fold(src, src_base, dst, dst_base, nbins, lane):
    @pl.loop(0, nbins // L)
    def _(i):
        bins = i * L + lane
        acc = jnp.zeros((L,), jnp.float32)
        for l in range(L):
            acc = acc + plsc.load_gather(src, [src_base + bins * L + l])
        dst[pl.ds(dst_base + i * L, L)] = acc


def _accum_rows(src_hbm, stage, acc, nwords, nrows):
    _zero_ref(acc, nwords)

    @pl.loop(0, nrows)
    def _(t):
        pltpu.sync_copy(src_hbm.at[pl.ds(t * nwords, nwords)], stage)

        @pl.loop(0, nwords // L)
        def _(i):
            sl = pl.ds(i * L, L)
            acc[sl] = acc[sl] + stage[sl]


def _combine_per_sc(fold_v, shared, stage, acc, out_hbm, nwords):
    sid = lax.axis_index("s")
    cid = lax.axis_index("c")
    pltpu.sync_copy(fold_v, shared.at[sid])
    plsc.subcore_barrier()

    @pl.when(sid == 0)
    def _():
        _zero_ref(acc, nwords)

        @pl.loop(0, NS)
        def _(t):
            pltpu.sync_copy(shared.at[t], stage)

            @pl.loop(0, nwords // L)
            def _(i):
                sl = pl.ds(i * L, L)
                acc[sl] = acc[sl] + stage[sl]

        pltpu.sync_copy(acc.at[pl.ds(0, nwords)],
                        out_hbm.at[pl.ds(cid * nwords, nwords)])


def _scan_top(ref, cnt_base, sum_base, nbins, target):
    nb = nbins // L

    def body(j, carry):
        found, b_star, s_above, c_above, ccnt, csum = carry
        vb = nb - 1 - j
        vc = ref[pl.ds(cnt_base + vb * L, L)]
        vs = ref[pl.ds(sum_base + vb * L, L)]
        rc = lax.rev(vc, (0,))
        rs = lax.rev(vs, (0,))
        cum = jnp.cumsum(rc) + ccnt
        m = cum >= target
        p = jnp.sum(jnp.where(m, 1.0, 0.0))
        has = (p > 0.5).astype(jnp.int32)
        b_here = vb * L + lax.convert_element_type(p, jnp.int32) - 1
        c_here = ccnt + jnp.sum(jnp.where(m, 0.0, rc))
        s_here = csum + jnp.sum(jnp.where(m, 0.0, rs))
        take = has * (1 - found)
        b_star = jnp.where(take > 0, b_here, b_star)
        s_above = jnp.where(take > 0, s_here, s_above)
        c_above = jnp.where(take > 0, c_here, c_above)
        found = jnp.maximum(found, has)
        ccnt = ccnt + jnp.sum(vc)
        csum = csum + jnp.sum(vs)
        return (found, b_star, s_above, c_above, ccnt, csum)

    init = (jnp.int32(0), jnp.int32(0), jnp.float32(0.0), jnp.float32(0.0),
            jnp.float32(0.0), jnp.float32(0.0))
    _, b_star, s_above, c_above, _, _ = lax.fori_loop(0, nb, body, init)
    return b_star, s_above, c_above


def _hist_count(ref, cnt_base, nbins):
    acc = jnp.zeros((L,), jnp.float32)

    def body(i, acc):
        return acc + ref[pl.ds(cnt_base + i * L, L)]

    acc = lax.fori_loop(0, nbins // L, body, acc)
    return jnp.sum(acc)


def _n_keep(count_f):
    ci = lax.convert_element_type(count_f, jnp.int32)
    nk = (7 * ci) // 10
    return lax.convert_element_type(nk, jnp.float32)


def _sdiv(a, b):
    q = jnp.full((L,), a, jnp.float32) / jnp.full((L,), b, jnp.float32)
    lane = lax.iota(jnp.int32, L)
    return jnp.sum(jnp.where(lane == 0, q, jnp.zeros((L,), jnp.float32)))


def _scalar_at(ref, idx):
    v = plsc.load_gather(ref, [jnp.full((L,), idx, jnp.int32)])
    return jnp.sum(v) * (1.0 / L)


@functools.partial(
    pl.kernel,
    out_type=jax.ShapeDtypeStruct((NC * 6 * B1,), jnp.float32),
    mesh=_MESH,
    compiler_params=_CPARAMS,
    scratch_types=(
        pltpu.VMEM((CH1,), jnp.float32),
        pltpu.VMEM((CH1,), jnp.float32),
        pltpu.VMEM((CH1,), jnp.float32),
        pltpu.VMEM((6 * B1 * L,), jnp.float32),
        pltpu.VMEM((6 * B1,), jnp.float32),
        pltpu.VMEM((6 * B1,), jnp.float32),
        pltpu.VMEM((6 * B1,), jnp.float32),
        pltpu.VMEM_SHARED((NS, 6 * B1), jnp.float32),
    ),
)
def _kernel_h1(vc_hbm, vo_hbm, vl_hbm, h1_hbm,
               bc_v, bo_v, bl_v, h_v, fold_v, stage_v, acc_v, shared):
    wid = _wid()
    lane = lax.iota(jnp.int32, L)
    ones = jnp.ones((L,), jnp.float32)
    c22 = jnp.full((L,), 22, jnp.int32)

    _zero_ref(h_v, 6 * B1 * L)

    @pl.loop(0, RW // CH1)
    def _(ci):
        row0 = wid * RW + ci * CH1
        pltpu.sync_copy(vc_hbm.at[pl.ds(row0, CH1)], bc_v)
        pltpu.sync_copy(vo_hbm.at[pl.ds(row0, CH1)], bo_v)
        pltpu.sync_copy(vl_hbm.at[pl.ds(row0, CH1)], bl_v)

        @pl.loop(0, CH1 // L)
        def _(g):
            sl = pl.ds(g * L, L)
            for k, ref in enumerate((bc_v, bo_v, bl_v)):
                v = ref[sl]
                m = v >= 0.0
                bits = plsc.bitcast(v, jnp.int32)
                b = lax.shift_right_logical(bits, c22)
                idx = (k * 2 * B1 + b) * L + lane
                plsc.addupdate_scatter(h_v, [idx], ones, mask=m)
                plsc.addupdate_scatter(h_v, [idx + B1 * L], v, mask=m)

    for k in range(6):
        _lane_fold(h_v, k * B1 * L, fold_v, k * B1, B1, lane)
    _combine_per_sc(fold_v, shared, stage_v, acc_v, h1_hbm, 6 * B1)


@functools.partial(
    pl.kernel,
    out_type=jax.ShapeDtypeStruct((NC * 6 * B2,), jnp.float32),
    mesh=_MESH,
    compiler_params=_CPARAMS,
    scratch_types=(
        pltpu.VMEM((6 * B1,), jnp.float32),
        pltpu.VMEM((6 * B1,), jnp.float32),
        pltpu.VMEM((CH2,), jnp.float32),
        pltpu.VMEM((CH2,), jnp.float32),
        pltpu.VMEM((CH2,), jnp.float32),
        pltpu.VMEM((6 * B2 * L,), jnp.float32),
        pltpu.VMEM((6 * B2,), jnp.float32),
        pltpu.VMEM_SHARED((NS, 6 * B2), jnp.float32),
    ),
)
def _kernel_h2(vc_hbm, vo_hbm, vl_hbm, h1_hbm, h2_hbm,
               acc1_v, st1_v, bc_v, bo_v, bl_v, h2_v, fold_v, shared):
    wid = _wid()
    lane = lax.iota(jnp.int32, L)
    ones = jnp.ones((L,), jnp.float32)
    c22 = jnp.full((L,), 22, jnp.int32)
    c12 = jnp.full((L,), 12, jnp.int32)

    _accum_rows(h1_hbm, st1_v, acc1_v, 6 * B1, NC)

    b1s = []
    for k in range(3):
        count = _hist_count(acc1_v, k * 2 * B1, B1)
        nk = _n_keep(count)
        b1, _, _ = _scan_top(acc1_v, k * 2 * B1, (k * 2 + 1) * B1, B1, nk)
        b1s.append(jnp.full((L,), b1, jnp.int32))

    _zero_ref(h2_v, 6 * B2 * L)

    @pl.loop(0, RW // CH2)
    def _(ci):
        row0 = wid * RW + ci * CH2
        pltpu.sync_copy(vc_hbm.at[pl.ds(row0, CH2)], bc_v)
        pltpu.sync_copy(vo_hbm.at[pl.ds(row0, CH2)], bo_v)
        pltpu.sync_copy(vl_hbm.at[pl.ds(row0, CH2)], bl_v)

        @pl.loop(0, CH2 // L)
        def _(g):
            sl = pl.ds(g * L, L)
            for k, ref in enumerate((bc_v, bo_v, bl_v)):
                v = ref[sl]
                bits = plsc.bitcast(v, jnp.int32)
                lvl1 = lax.shift_right_logical(bits, c22)
                m = lvl1 == b1s[k]
                sub = jnp.bitwise_and(lax.shift_right_logical(bits, c12),
                                      B2 - 1)
                idx = (k * 2 * B2 + sub) * L + lane
                plsc.addupdate_scatter(h2_v, [idx], ones, mask=m)
                plsc.addupdate_scatter(h2_v, [idx + B2 * L], v, mask=m)

    for k in range(6):
        _lane_fold(h2_v, k * B2 * L, fold_v, k * B2, B2, lane)
    _combine_per_sc(fold_v, shared, fold_v, h2_v, h2_hbm, 6 * B2)


@functools.partial(
    pl.kernel,
    out_type=jax.ShapeDtypeStruct((8,), jnp.float32),
    mesh=_MESH,
    compiler_params=_CPARAMS,
    scratch_types=(
        pltpu.VMEM((6 * B1,), jnp.float32),
        pltpu.VMEM((6 * B2,), jnp.float32),
        pltpu.VMEM((6 * B1,), jnp.float32),
        pltpu.VMEM((6 * B2,), jnp.float32),
        pltpu.VMEM((16,), jnp.float32),
    ),
)
def _kernel_c(h1_hbm, h2_hbm, out_hbm, acc1_v, acc2_v, st1_v, st2_v, out_v):
    wid = _wid()

    @pl.when(wid == 0)
    def _():
        _accum_rows(h1_hbm, st1_v, acc1_v, 6 * B1, NC)
        _accum_rows(h2_hbm, st2_v, acc2_v, 6 * B2, NC)

        losses = []
        for k in range(3):
            count = _hist_count(acc1_v, k * 2 * B1, B1)
            nk = _n_keep(count)
            _, s1, c1 = _scan_top(acc1_v, k * 2 * B1, (k * 2 + 1) * B1,
                                  B1, nk)
            r1 = nk - c1
            b2, s2, c2 = _scan_top(acc2_v, k * 2 * B2, (k * 2 + 1) * B2,
                                   B2, r1)
            r2 = r1 - c2
            cnt_b2 = _scalar_at(acc2_v, k * 2 * B2 + b2)
            sum_b2 = _scalar_at(acc2_v, (k * 2 + 1) * B2 + b2)
            part = jnp.where(r2 > 0.5, r2 * _sdiv(sum_b2, cnt_b2), 0.0)
            total = s1 + s2 + part
            mean = _sdiv(total, nk)
            if k == 0:
                losses.append(mean)
            else:
                losses.append(jnp.where(count < 0.5, 0.0, mean))

        loss_cls, loss_off, loss_lmk = losses
        loss_total = CLS_W * loss_cls + BBOX_W * loss_off + LMK_W * loss_lmk
        lane = lax.iota(jnp.int32, L)
        zeros = jnp.zeros((L,), jnp.float32)
        ov = jnp.where(lane == 0, loss_total, zeros)
        ov = ov + jnp.where(lane == 1, loss_cls, zeros)
        ov = ov + jnp.where(lane == 2, loss_off, zeros)
        ov = ov + jnp.where(lane == 3, loss_lmk, zeros)
        out_v[pl.ds(0, L)] = ov
        pltpu.sync_copy(out_v.at[pl.ds(0, 8)], out_hbm)


def kernel(pred, labels, offsets, landmarks):
    vc, vo, vl = _tc_values(pred, labels, offsets, landmarks)
    h1 = _kernel_h1(vc, vo, vl)
    h2 = _kernel_h2(vc, vo, vl, h1)
    out = _kernel_c(h1, h2)
    return (out[0], out[1], out[2], out[3])


# speedup vs baseline: 2.0505x; 1.4699x over previous
import functools

import jax
import jax.numpy as jnp
import numpy as np
from jax import lax
from jax.experimental import pallas as pl
from jax.experimental.pallas import tpu as pltpu
from jax.experimental.pallas import tpu_sc as plsc

N = 1048576
NC = 2
NS = 16
NW = NC * NS
L = 16
RW = N // NW

RTC = 4096
CH = 2048
NCH = RW // CH

B1 = 512
B2 = 512

CLS_W = 1.0
BBOX_W = 0.5
LMK_W = 0.5

_G_COEF = (1.8498544538905285e-04, 2.8751506391739456e-04,
           -5.4268610571399910e-03, 8.3107776364009530e-05,
           1.2498464620813230e-01, -4.9999884358222030e-01,
           6.9314715967354310e-01)

_MESH = plsc.VectorSubcoreMesh(core_axis_name="c", subcore_axis_name="s")
_CPARAMS = pltpu.CompilerParams(needs_layout_passes=False)


def _dot(a, b):
    return jax.lax.dot_general(
        a, b, (((1,), (0,)), ((), ())),
        preferred_element_type=jnp.float32)


def _sel(rows, cols, fn):
    r = lax.broadcasted_iota(jnp.int32, (rows, cols), 0)
    c = lax.broadcasted_iota(jnp.int32, (rows, cols), 1)
    return fn(r, c).astype(jnp.float32)


def _tc_body(pred_ref, off_ref, lmk_ref, v_ref):
    p = pred_ref[...]
    off = off_ref[...]
    lmk = lmk_ref[...]
    e0 = _sel(15, 8, lambda r, c: (r == 0) & (c == 0))
    s12 = _sel(15, 8, lambda r, c: ((c == 1) & (r >= 1) & (r <= 4))
               | ((c == 2) & (r >= 5)))
    t = jnp.pad(off, ((0, 0), (1, 10))) + jnp.pad(lmk, ((0, 0), (5, 0)))
    d = p - t
    v_ref[...] = _dot(p, e0) + _dot(d * d, s12)


_tc_values = pl.pallas_call(
    _tc_body,
    grid=(N // RTC,),
    in_specs=[
        pl.BlockSpec((RTC, 15), lambda i: (i, 0)),
        pl.BlockSpec((RTC, 4), lambda i: (i, 0)),
        pl.BlockSpec((RTC, 10), lambda i: (i, 0)),
    ],
    out_specs=pl.BlockSpec((RTC, 8), lambda i: (i, 0)),
    out_shape=jax.ShapeDtypeStruct((N, 8), jnp.float32),
)


def _wid():
    return lax.axis_index("s") * NC + lax.axis_index("c")


def _g_poly(s):
    acc = jnp.full(s.shape, _G_COEF[0], jnp.float32)
    for c in _G_COEF[1:]:
        acc = acc * s + c
    return acc


def _zero_ref(ref, nwords):
    z = jnp.zeros((L,), jnp.float32)

    @pl.loop(0, nwords // L)
    def _(i):
        ref[pl.ds(i * L, L)] = z


def _lane_fold(src, src_base, dst, dst_base, nbins, lane):
    @pl.loop(0, nbins // L)
    def _(i):
        bins = i * L + lane
        acc = jnp.zeros((L,), jnp.float32)
        for l in range(L):
            acc = acc + plsc.load_gather(src, [src_base + bins * L + l])
        dst[pl.ds(dst_base + i * L, L)] = acc


def _accum_rows(src_hbm, stage, acc, nwords, nrows):
    _zero_ref(acc, nwords)

    @pl.loop(0, nrows)
    def _(t):
        pltpu.sync_copy(src_hbm.at[pl.ds(t * nwords, nwords)], stage)

        @pl.loop(0, nwords // L)
        def _(i):
            sl = pl.ds(i * L, L)
            acc[sl] = acc[sl] + stage[sl]


def _combine_per_sc(fold_v, shared, stage, acc, out_hbm, nwords):
    sid = lax.axis_index("s")
    cid = lax.axis_index("c")
    pltpu.sync_copy(fold_v, shared.at[sid])
    plsc.subcore_barrier()

    @pl.when(sid == 0)
    def _():
        _zero_ref(acc, nwords)

        @pl.loop(0, NS)
        def _(t):
            pltpu.sync_copy(shared.at[t], stage)

            @pl.loop(0, nwords // L)
            def _(i):
                sl = pl.ds(i * L, L)
                acc[sl] = acc[sl] + stage[sl]

        pltpu.sync_copy(acc.at[pl.ds(0, nwords)],
                        out_hbm.at[pl.ds(cid * nwords, nwords)])


def _scan_top(ref, cnt_base, sum_base, nbins, target):
    nb = nbins // L

    def body(j, carry):
        found, b_star, s_above, c_above, ccnt, csum = carry
        vb = nb - 1 - j
        vc = ref[pl.ds(cnt_base + vb * L, L)]
        vs = ref[pl.ds(sum_base + vb * L, L)]
        rc = lax.rev(vc, (0,))
        rs = lax.rev(vs, (0,))
        cum = jnp.cumsum(rc) + ccnt
        m = cum >= target
        p = jnp.sum(jnp.where(m, 1.0, 0.0))
        has = (p > 0.5).astype(jnp.int32)
        b_here = vb * L + lax.convert_element_type(p, jnp.int32) - 1
        c_here = ccnt + jnp.sum(jnp.where(m, 0.0, rc))
        s_here = csum + jnp.sum(jnp.where(m, 0.0, rs))
        take = has * (1 - found)
        b_star = jnp.where(take > 0, b_here, b_star)
        s_above = jnp.where(take > 0, s_here, s_above)
        c_above = jnp.where(take > 0, c_here, c_above)
        found = jnp.maximum(found, has)
        ccnt = ccnt + jnp.sum(vc)
        csum = csum + jnp.sum(vs)
        return (found, b_star, s_above, c_above, ccnt, csum)

    init = (jnp.int32(0), jnp.int32(0), jnp.float32(0.0), jnp.float32(0.0),
            jnp.float32(0.0), jnp.float32(0.0))
    _, b_star, s_above, c_above, _, _ = lax.fori_loop(0, nb, body, init)
    return b_star, s_above, c_above


def _hist_count(ref, cnt_base, nbins):
    acc = jnp.zeros((L,), jnp.float32)

    def body(i, acc):
        return acc + ref[pl.ds(cnt_base + i * L, L)]

    acc = lax.fori_loop(0, nbins // L, body, acc)
    return jnp.sum(acc)


def _n_keep(count_f):
    ci = lax.convert_element_type(count_f, jnp.int32)
    nk = (7 * ci) // 10
    return lax.convert_element_type(nk, jnp.float32)


def _sdiv(a, b):
    q = jnp.full((L,), a, jnp.float32) / jnp.full((L,), b, jnp.float32)
    lane = lax.iota(jnp.int32, L)
    return jnp.sum(jnp.where(lane == 0, q, jnp.zeros((L,), jnp.float32)))


def _scalar_at(ref, idx):
    v = plsc.load_gather(ref, [jnp.full((L,), idx, jnp.int32)])
    return jnp.sum(v) * (1.0 / L)


@functools.partial(
    pl.kernel,
    out_type=(
        jax.ShapeDtypeStruct((N,), jnp.float32),
        jax.ShapeDtypeStruct((N,), jnp.float32),
        jax.ShapeDtypeStruct((N,), jnp.float32),
        jax.ShapeDtypeStruct((NC * 6 * B1,), jnp.float32),
    ),
    mesh=_MESH,
    compiler_params=_CPARAMS,
    scratch_types=(
        (pltpu.VMEM((CH,), jnp.int32),) * 2,
        (pltpu.VMEM((CH * 8,), jnp.float32),) * 2,
        (pltpu.VMEM((CH,), jnp.float32),) * 2,
        (pltpu.VMEM((CH,), jnp.float32),) * 2,
        (pltpu.VMEM((CH,), jnp.float32),) * 2,
        pltpu.VMEM((6 * B1 * L,), jnp.float32),
        pltpu.VMEM((6 * B1,), jnp.float32),
        pltpu.VMEM((6 * B1,), jnp.float32),
        pltpu.VMEM((6 * B1,), jnp.float32),
        pltpu.VMEM_SHARED((NS, 6 * B1), jnp.float32),
        (pltpu.SemaphoreType.DMA,) * 2,
        (pltpu.SemaphoreType.DMA,) * 2,
    ),
)
def _kernel_h1(lab_hbm, v8_hbm, vc_hbm, vo_hbm, vl_hbm, h1_hbm,
               lab_b, v8_b, oc_b, oo_b, ol_b,
               h_v, fold_v, stage_v, acc_v, shared, semi, semo):
    wid = _wid()
    lane = lax.iota(jnp.int32, L)
    ones = jnp.ones((L,), jnp.float32)
    neg1 = jnp.full((L,), -1.0, jnp.float32)
    c22 = jnp.full((L,), 22, jnp.int32)

    def start_in(ci, b):
        row0 = wid * RW + ci * CH
        pltpu.async_copy(lab_hbm.at[pl.ds(row0, CH)], lab_b[b], semi[b])
        pltpu.async_copy(v8_hbm.at[pl.ds(row0 * 8, CH * 8)], v8_b[b],
                         semi[b])

    def wait_in(b):
        pltpu.make_async_copy(lab_hbm.at[pl.ds(0, CH)], lab_b[b],
                              semi[b]).wait()
        pltpu.make_async_copy(v8_hbm.at[pl.ds(0, CH * 8)], v8_b[b],
                              semi[b]).wait()

    def start_out(ci, b):
        row0 = wid * RW + ci * CH
        pltpu.async_copy(oc_b[b], vc_hbm.at[pl.ds(row0, CH)], semo[b])
        pltpu.async_copy(oo_b[b], vo_hbm.at[pl.ds(row0, CH)], semo[b])
        pltpu.async_copy(ol_b[b], vl_hbm.at[pl.ds(row0, CH)], semo[b])

    def wait_out(b):
        for buf, hbm in ((oc_b, vc_hbm), (oo_b, vo_hbm), (ol_b, vl_hbm)):
            pltpu.make_async_copy(buf[b], hbm.at[pl.ds(0, CH)],
                                  semo[b]).wait()

    _zero_ref(h_v, 6 * B1 * L)
    start_in(0, 0)

    @pl.loop(0, NCH // 2)
    def _(oc):
        for b in range(2):
            ci = oc * 2 + b
            wait_in(b)

            @pl.when(ci + 1 < NCH)
            def _():
                start_in(ci + 1, 1 - b)

            @pl.when(ci >= 2)
            def _():
                wait_out(b)

            @pl.loop(0, CH // L)
            def _(g):
                sl = pl.ds(g * L, L)
                rows = g * L + lane
                lbl = lab_b[b][sl]
                z = plsc.load_gather(v8_b[b], [rows * 8])
                so = plsc.load_gather(v8_b[b], [rows * 8 + 1])
                sl10 = plsc.load_gather(v8_b[b], [rows * 8 + 2])

                s = 1.0 / (1.0 + jnp.exp(-z))
                y = jnp.where(lbl == 1, 1.0, 0.0)
                per_cls = s * (1.0 - y) + _g_poly(s)
                keep = lbl >= 0
                per_off = so * 0.25
                offm = (lbl == 1) | (lbl == -1)
                per_lmk = sl10 * 0.1
                lmkm = lbl == -2

                oc_b[b][sl] = jnp.where(keep, per_cls, neg1)
                oo_b[b][sl] = jnp.where(offm, per_off, neg1)
                ol_b[b][sl] = jnp.where(lmkm, per_lmk, neg1)

                for k, (per, msk) in enumerate(
                        ((per_cls, keep), (per_off, offm),
                         (per_lmk, lmkm))):
                    bits = plsc.bitcast(per, jnp.int32)
                    bb = lax.shift_right_logical(bits, c22)
                    idx = (k * 2 * B1 + bb) * L + lane
                    plsc.addupdate_scatter(h_v, [idx], ones, mask=msk)
                    plsc.addupdate_scatter(h_v, [idx + B1 * L], per,
                                           mask=msk)

            start_out(ci, b)

    for b in range(2):
        wait_out(b)

    for k in range(6):
        _lane_fold(h_v, k * B1 * L, fold_v, k * B1, B1, lane)
    _combine_per_sc(fold_v, shared, stage_v, acc_v, h1_hbm, 6 * B1)


@functools.partial(
    pl.kernel,
    out_type=jax.ShapeDtypeStruct((NC * 6 * B2,), jnp.float32),
    mesh=_MESH,
    compiler_params=_CPARAMS,
    scratch_types=(
        pltpu.VMEM((6 * B1,), jnp.float32),
        pltpu.VMEM((6 * B1,), jnp.float32),
        (pltpu.VMEM((CH,), jnp.float32),) * 2,
        (pltpu.VMEM((CH,), jnp.float32),) * 2,
        (pltpu.VMEM((CH,), jnp.float32),) * 2,
        pltpu.VMEM((6 * B2 * L,), jnp.float32),
        pltpu.VMEM((6 * B2,), jnp.float32),
        pltpu.VMEM_SHARED((NS, 6 * B2), jnp.float32),
        (pltpu.SemaphoreType.DMA,) * 2,
    ),
)
def _kernel_h2(vc_hbm, vo_hbm, vl_hbm, h1_hbm, h2_hbm,
               acc1_v, st1_v, bc_b, bo_b, bl_b, h2_v, fold_v, shared, semi):
    wid = _wid()
    lane = lax.iota(jnp.int32, L)
    ones = jnp.ones((L,), jnp.float32)
    c22 = jnp.full((L,), 22, jnp.int32)
    c13 = jnp.full((L,), 13, jnp.int32)

    def start_in(ci, b):
        row0 = wid * RW + ci * CH
        for buf, hbm in ((bc_b, vc_hbm), (bo_b, vo_hbm), (bl_b, vl_hbm)):
            pltpu.async_copy(hbm.at[pl.ds(row0, CH)], buf[b], semi[b])

    def wait_in(b):
        for buf, hbm in ((bc_b, vc_hbm), (bo_b, vo_hbm), (bl_b, vl_hbm)):
            pltpu.make_async_copy(hbm.at[pl.ds(0, CH)], buf[b],
                                  semi[b]).wait()

    _accum_rows(h1_hbm, st1_v, acc1_v, 6 * B1, NC)

    b1s = []
    for k in range(3):
        count = _hist_count(acc1_v, k * 2 * B1, B1)
        nk = _n_keep(count)
        b1, _, _ = _scan_top(acc1_v, k * 2 * B1, (k * 2 + 1) * B1, B1, nk)
        b1s.append(jnp.full((L,), b1, jnp.int32))

    _zero_ref(h2_v, 6 * B2 * L)
    start_in(0, 0)

    @pl.loop(0, NCH // 2)
    def _(oc):
        for b in range(2):
            ci = oc * 2 + b
            wait_in(b)

            @pl.when(ci + 1 < NCH)
            def _():
                start_in(ci + 1, 1 - b)

            @pl.loop(0, CH // L)
            def _(g):
                sl = pl.ds(g * L, L)
                for k, bufs in enumerate((bc_b, bo_b, bl_b)):
                    v = bufs[b][sl]
                    bits = plsc.bitcast(v, jnp.int32)
                    lvl1 = lax.shift_right_logical(bits, c22)
                    m = lvl1 == b1s[k]
                    sub = jnp.bitwise_and(
                        lax.shift_right_logical(bits, c13), B2 - 1)
                    idx = (k * 2 * B2 + sub) * L + lane
                    plsc.addupdate_scatter(h2_v, [idx], ones, mask=m)
                    plsc.addupdate_scatter(h2_v, [idx + B2 * L], v, mask=m)

    for k in range(6):
        _lane_fold(h2_v, k * B2 * L, fold_v, k * B2, B2, lane)
    _combine_per_sc(fold_v, shared, fold_v, h2_v, h2_hbm, 6 * B2)


@functools.partial(
    pl.kernel,
    out_type=jax.ShapeDtypeStruct((8,), jnp.float32),
    mesh=_MESH,
    compiler_params=_CPARAMS,
    scratch_types=(
        pltpu.VMEM((6 * B1,), jnp.float32),
        pltpu.VMEM((6 * B2,), jnp.float32),
        pltpu.VMEM((6 * B1,), jnp.float32),
        pltpu.VMEM((6 * B2,), jnp.float32),
        pltpu.VMEM((16,), jnp.float32),
    ),
)
def _kernel_c(h1_hbm, h2_hbm, out_hbm, acc1_v, acc2_v, st1_v, st2_v, out_v):
    wid = _wid()

    @pl.when(wid == 0)
    def _():
        _accum_rows(h1_hbm, st1_v, acc1_v, 6 * B1, NC)
        _accum_rows(h2_hbm, st2_v, acc2_v, 6 * B2, NC)

        losses = []
        for k in range(3):
            count = _hist_count(acc1_v, k * 2 * B1, B1)
            nk = _n_keep(count)
            _, s1, c1 = _scan_top(acc1_v, k * 2 * B1, (k * 2 + 1) * B1,
                                  B1, nk)
            r1 = nk - c1
            b2, s2, c2 = _scan_top(acc2_v, k * 2 * B2, (k * 2 + 1) * B2,
                                   B2, r1)
            r2 = r1 - c2
            cnt_b2 = _scalar_at(acc2_v, k * 2 * B2 + b2)
            sum_b2 = _scalar_at(acc2_v, (k * 2 + 1) * B2 + b2)
            part = jnp.where(r2 > 0.5, r2 * _sdiv(sum_b2, cnt_b2), 0.0)
            total = s1 + s2 + part
            mean = _sdiv(total, nk)
            if k == 0:
                losses.append(mean)
            else:
                losses.append(jnp.where(count < 0.5, 0.0, mean))

        loss_cls, loss_off, loss_lmk = losses
        loss_total = CLS_W * loss_cls + BBOX_W * loss_off + LMK_W * loss_lmk
        lane = lax.iota(jnp.int32, L)
        zeros = jnp.zeros((L,), jnp.float32)
        ov = jnp.where(lane == 0, loss_total, zeros)
        ov = ov + jnp.where(lane == 1, loss_cls, zeros)
        ov = ov + jnp.where(lane == 2, loss_off, zeros)
        ov = ov + jnp.where(lane == 3, loss_lmk, zeros)
        out_v[pl.ds(0, L)] = ov
        pltpu.sync_copy(out_v.at[pl.ds(0, 8)], out_hbm)


def kernel(pred, labels, offsets, landmarks):
    v8 = _tc_values(pred, offsets, landmarks)
    vc, vo, vl, h1 = _kernel_h1(labels, v8.reshape(-1))
    h2 = _kernel_h2(vc, vo, vl, h1)
    out = _kernel_c(h1, h2)
    return (out[0], out[1], out[2], out[3])


# speedup vs baseline: 12.1635x; 5.9319x over previous
import functools

import jax
import jax.numpy as jnp
import numpy as np
from jax import lax
from jax.experimental import pallas as pl
from jax.experimental.pallas import tpu as pltpu
from jax.experimental.pallas import tpu_sc as plsc

N = 1048576
NC = 2
NS = 16
NW = NC * NS
L = 16
RW = N // NW

RTC = 16384
CH = 2048
NCH = RW // CH

B1 = 512
B2 = 512

CLS_W = 1.0
BBOX_W = 0.5
LMK_W = 0.5

_G_COEF = (1.8498544538905285e-04, 2.8751506391739456e-04,
           -5.4268610571399910e-03, 8.3107776364009530e-05,
           1.2498464620813230e-01, -4.9999884358222030e-01,
           6.9314715967354310e-01)

_MESH = plsc.VectorSubcoreMesh(core_axis_name="c", subcore_axis_name="s")
_CPARAMS = pltpu.CompilerParams(needs_layout_passes=False)


def _tc_body(pred_ref, off_ref, lmk_ref, vz_ref, vo_ref, vl_ref):
    pt = pred_ref[...]
    ot = off_ref[...]
    lt = lmk_ref[...]
    do = pt[1:5, :] - ot
    dl = pt[5:15, :] - lt
    vz_ref[...] = pt[0, :]
    vo_ref[...] = jnp.sum(do * do, axis=0)
    vl_ref[...] = jnp.sum(dl * dl, axis=0)


_tc_values = pl.pallas_call(
    _tc_body,
    grid=(N // RTC,),
    in_specs=[
        pl.BlockSpec((15, RTC), lambda i: (0, i)),
        pl.BlockSpec((4, RTC), lambda i: (0, i)),
        pl.BlockSpec((10, RTC), lambda i: (0, i)),
    ],
    out_specs=[pl.BlockSpec((RTC,), lambda i: (i,))] * 3,
    out_shape=[jax.ShapeDtypeStruct((N,), jnp.float32)] * 3,
)


def _wid():
    return lax.axis_index("s") * NC + lax.axis_index("c")


def _g_poly(s):
    acc = jnp.full(s.shape, _G_COEF[0], jnp.float32)
    for c in _G_COEF[1:]:
        acc = acc * s + c
    return acc


def _zero_ref(ref, nwords):
    z = jnp.zeros((L,), jnp.float32)

    @pl.loop(0, nwords // L)
    def _(i):
        ref[pl.ds(i * L, L)] = z


def _lane_fold(src, src_base, dst, dst_base, nbins, lane):
    @pl.loop(0, nbins // L)
    def _(i):
        bins = i * L + lane
        acc = jnp.zeros((L,), jnp.float32)
        for l in range(L):
            acc = acc + plsc.load_gather(src, [src_base + bins * L + l])
        dst[pl.ds(dst_base + i * L, L)] = acc


def _accum_rows(src_hbm, stage, acc, nwords, nrows):
    _zero_ref(acc, nwords)

    @pl.loop(0, nrows)
    def _(t):
        pltpu.sync_copy(src_hbm.at[pl.ds(t * nwords, nwords)], stage)

        @pl.loop(0, nwords // L)
        def _(i):
            sl = pl.ds(i * L, L)
            acc[sl] = acc[sl] + stage[sl]


def _combine_per_sc(fold_v, shared, stage, acc, out_hbm, nwords):
    sid = lax.axis_index("s")
    cid = lax.axis_index("c")
    pltpu.sync_copy(fold_v, shared.at[sid])
    plsc.subcore_barrier()

    @pl.when(sid == 0)
    def _():
        _zero_ref(acc, nwords)

        @pl.loop(0, NS)
        def _(t):
            pltpu.sync_copy(shared.at[t], stage)

            @pl.loop(0, nwords // L)
            def _(i):
                sl = pl.ds(i * L, L)
                acc[sl] = acc[sl] + stage[sl]

        pltpu.sync_copy(acc.at[pl.ds(0, nwords)],
                        out_hbm.at[pl.ds(cid * nwords, nwords)])


def _scan_top(ref, cnt_base, sum_base, nbins, target):
    nb = nbins // L

    def body(j, carry):
        found, b_star, s_above, c_above, ccnt, csum = carry
        vb = nb - 1 - j
        vc = ref[pl.ds(cnt_base + vb * L, L)]
        vs = ref[pl.ds(sum_base + vb * L, L)]
        rc = lax.rev(vc, (0,))
        rs = lax.rev(vs, (0,))
        cum = jnp.cumsum(rc) + ccnt
        m = cum >= target
        p = jnp.sum(jnp.where(m, 1.0, 0.0))
        has = (p > 0.5).astype(jnp.int32)
        b_here = vb * L + lax.convert_element_type(p, jnp.int32) - 1
        c_here = ccnt + jnp.sum(jnp.where(m, 0.0, rc))
        s_here = csum + jnp.sum(jnp.where(m, 0.0, rs))
        take = has * (1 - found)
        b_star = jnp.where(take > 0, b_here, b_star)
        s_above = jnp.where(take > 0, s_here, s_above)
        c_above = jnp.where(take > 0, c_here, c_above)
        found = jnp.maximum(found, has)
        ccnt = ccnt + jnp.sum(vc)
        csum = csum + jnp.sum(vs)
        return (found, b_star, s_above, c_above, ccnt, csum)

    init = (jnp.int32(0), jnp.int32(0), jnp.float32(0.0), jnp.float32(0.0),
            jnp.float32(0.0), jnp.float32(0.0))
    _, b_star, s_above, c_above, _, _ = lax.fori_loop(0, nb, body, init)
    return b_star, s_above, c_above


def _hist_count(ref, cnt_base, nbins):
    acc = jnp.zeros((L,), jnp.float32)

    def body(i, acc):
        return acc + ref[pl.ds(cnt_base + i * L, L)]

    acc = lax.fori_loop(0, nbins // L, body, acc)
    return jnp.sum(acc)


def _n_keep(count_f):
    ci = lax.convert_element_type(count_f, jnp.int32)
    nk = (7 * ci) // 10
    return lax.convert_element_type(nk, jnp.float32)


def _sdiv(a, b):
    q = jnp.full((L,), a, jnp.float32) / jnp.full((L,), b, jnp.float32)
    lane = lax.iota(jnp.int32, L)
    return jnp.sum(jnp.where(lane == 0, q, jnp.zeros((L,), jnp.float32)))


def _scalar_at(ref, idx):
    v = plsc.load_gather(ref, [jnp.full((L,), idx, jnp.int32)])
    return jnp.sum(v) * (1.0 / L)


@functools.partial(
    pl.kernel,
    out_type=(
        jax.ShapeDtypeStruct((N,), jnp.float32),
        jax.ShapeDtypeStruct((N,), jnp.float32),
        jax.ShapeDtypeStruct((N,), jnp.float32),
        jax.ShapeDtypeStruct((NC * 6 * B1,), jnp.float32),
    ),
    mesh=_MESH,
    compiler_params=_CPARAMS,
    scratch_types=(
        (pltpu.VMEM((CH,), jnp.int32),) * 2,
        (pltpu.VMEM((CH,), jnp.float32),) * 2,
        (pltpu.VMEM((CH,), jnp.float32),) * 2,
        (pltpu.VMEM((CH,), jnp.float32),) * 2,
        (pltpu.VMEM((CH,), jnp.float32),) * 2,
        (pltpu.VMEM((CH,), jnp.float32),) * 2,
        (pltpu.VMEM((CH,), jnp.float32),) * 2,
        pltpu.VMEM((6 * B1 * L,), jnp.float32),
        pltpu.VMEM((6 * B1,), jnp.float32),
        pltpu.VMEM((6 * B1,), jnp.float32),
        pltpu.VMEM((6 * B1,), jnp.float32),
        pltpu.VMEM_SHARED((NS, 6 * B1), jnp.float32),
        (pltpu.SemaphoreType.DMA,) * 2,
        (pltpu.SemaphoreType.DMA,) * 2,
    ),
)
def _kernel_h1(lab_hbm, vz_hbm, vso_hbm, vsl_hbm,
               vc_hbm, vo_hbm, vl_hbm, h1_hbm,
               lab_b, z_b, so_b, sl_b, oc_b, oo_b, ol_b,
               h_v, fold_v, stage_v, acc_v, shared, semi, semo):
    wid = _wid()
    lane = lax.iota(jnp.int32, L)
    ones = jnp.ones((L,), jnp.float32)
    neg1 = jnp.full((L,), -1.0, jnp.float32)
    c22 = jnp.full((L,), 22, jnp.int32)

    in_pairs = ((lab_hbm, lab_b), (vz_hbm, z_b), (vso_hbm, so_b),
                (vsl_hbm, sl_b))

    def start_in(ci, b):
        row0 = wid * RW + ci * CH
        for hbm, buf in in_pairs:
            pltpu.async_copy(hbm.at[pl.ds(row0, CH)], buf[b], semi[b])

    def wait_in(b):
        for hbm, buf in in_pairs:
            pltpu.make_async_copy(hbm.at[pl.ds(0, CH)], buf[b],
                                  semi[b]).wait()

    def start_out(ci, b):
        row0 = wid * RW + ci * CH
        pltpu.async_copy(oc_b[b], vc_hbm.at[pl.ds(row0, CH)], semo[b])
        pltpu.async_copy(oo_b[b], vo_hbm.at[pl.ds(row0, CH)], semo[b])
        pltpu.async_copy(ol_b[b], vl_hbm.at[pl.ds(row0, CH)], semo[b])

    def wait_out(b):
        for buf, hbm in ((oc_b, vc_hbm), (oo_b, vo_hbm), (ol_b, vl_hbm)):
            pltpu.make_async_copy(buf[b], hbm.at[pl.ds(0, CH)],
                                  semo[b]).wait()

    _zero_ref(h_v, 6 * B1 * L)
    start_in(0, 0)

    @pl.loop(0, NCH // 2)
    def _(oc):
        for b in range(2):
            ci = oc * 2 + b
            wait_in(b)

            @pl.when(ci + 1 < NCH)
            def _():
                start_in(ci + 1, 1 - b)

            @pl.when(ci >= 2)
            def _():
                wait_out(b)

            @pl.loop(0, CH // L)
            def _(g):
                sl = pl.ds(g * L, L)
                lbl = lab_b[b][sl]
                z = z_b[b][sl]
                so = so_b[b][sl]
                sl10 = sl_b[b][sl]

                s = 1.0 / (1.0 + jnp.exp(-z))
                y = jnp.where(lbl == 1, 1.0, 0.0)
                per_cls = s * (1.0 - y) + _g_poly(s)
                keep = lbl >= 0
                per_off = so * 0.25
                offm = (lbl == 1) | (lbl == -1)
                per_lmk = sl10 * 0.1
                lmkm = lbl == -2

                oc_b[b][sl] = jnp.where(keep, per_cls, neg1)
                oo_b[b][sl] = jnp.where(offm, per_off, neg1)
                ol_b[b][sl] = jnp.where(lmkm, per_lmk, neg1)

                for k, (per, msk) in enumerate(
                        ((per_cls, keep), (per_off, offm),
                         (per_lmk, lmkm))):
                    bits = plsc.bitcast(per, jnp.int32)
                    bb = lax.shift_right_logical(bits, c22)
                    idx = (k * 2 * B1 + bb) * L + lane
                    plsc.addupdate_scatter(h_v, [idx], ones, mask=msk)
                    plsc.addupdate_scatter(h_v, [idx + B1 * L], per,
                                           mask=msk)

            start_out(ci, b)

    for b in range(2):
        wait_out(b)

    for k in range(6):
        _lane_fold(h_v, k * B1 * L, fold_v, k * B1, B1, lane)
    _combine_per_sc(fold_v, shared, stage_v, acc_v, h1_hbm, 6 * B1)


@functools.partial(
    pl.kernel,
    out_type=jax.ShapeDtypeStruct((NC * 6 * B2,), jnp.float32),
    mesh=_MESH,
    compiler_params=_CPARAMS,
    scratch_types=(
        pltpu.VMEM((6 * B1,), jnp.float32),
        pltpu.VMEM((6 * B1,), jnp.float32),
        (pltpu.VMEM((CH,), jnp.float32),) * 2,
        (pltpu.VMEM((CH,), jnp.float32),) * 2,
        (pltpu.VMEM((CH,), jnp.float32),) * 2,
        pltpu.VMEM((6 * B2 * L,), jnp.float32),
        pltpu.VMEM((6 * B2,), jnp.float32),
        pltpu.VMEM_SHARED((NS, 6 * B2), jnp.float32),
        (pltpu.SemaphoreType.DMA,) * 2,
    ),
)
def _kernel_h2(vc_hbm, vo_hbm, vl_hbm, h1_hbm, h2_hbm,
               acc1_v, st1_v, bc_b, bo_b, bl_b, h2_v, fold_v, shared, semi):
    wid = _wid()
    lane = lax.iota(jnp.int32, L)
    ones = jnp.ones((L,), jnp.float32)
    c22 = jnp.full((L,), 22, jnp.int32)
    c13 = jnp.full((L,), 13, jnp.int32)

    def start_in(ci, b):
        row0 = wid * RW + ci * CH
        for buf, hbm in ((bc_b, vc_hbm), (bo_b, vo_hbm), (bl_b, vl_hbm)):
            pltpu.async_copy(hbm.at[pl.ds(row0, CH)], buf[b], semi[b])

    def wait_in(b):
        for buf, hbm in ((bc_b, vc_hbm), (bo_b, vo_hbm), (bl_b, vl_hbm)):
            pltpu.make_async_copy(hbm.at[pl.ds(0, CH)], buf[b],
                                  semi[b]).wait()

    _accum_rows(h1_hbm, st1_v, acc1_v, 6 * B1, NC)

    b1s = []
    for k in range(3):
        count = _hist_count(acc1_v, k * 2 * B1, B1)
        nk = _n_keep(count)
        b1, _, _ = _scan_top(acc1_v, k * 2 * B1, (k * 2 + 1) * B1, B1, nk)
        b1s.append(jnp.full((L,), b1, jnp.int32))

    _zero_ref(h2_v, 6 * B2 * L)
    start_in(0, 0)

    @pl.loop(0, NCH // 2)
    def _(oc):
        for b in range(2):
            ci = oc * 2 + b
            wait_in(b)

            @pl.when(ci + 1 < NCH)
            def _():
                start_in(ci + 1, 1 - b)

            @pl.loop(0, CH // L)
            def _(g):
                sl = pl.ds(g * L, L)
                for k, bufs in enumerate((bc_b, bo_b, bl_b)):
                    v = bufs[b][sl]
                    bits = plsc.bitcast(v, jnp.int32)
                    lvl1 = lax.shift_right_logical(bits, c22)
                    m = lvl1 == b1s[k]
                    sub = jnp.bitwise_and(
                        lax.shift_right_logical(bits, c13), B2 - 1)
                    idx = (k * 2 * B2 + sub) * L + lane
                    plsc.addupdate_scatter(h2_v, [idx], ones, mask=m)
                    plsc.addupdate_scatter(h2_v, [idx + B2 * L], v, mask=m)

    for k in range(6):
        _lane_fold(h2_v, k * B2 * L, fold_v, k * B2, B2, lane)
    _combine_per_sc(fold_v, shared, fold_v, h2_v, h2_hbm, 6 * B2)


@functools.partial(
    pl.kernel,
    out_type=jax.ShapeDtypeStruct((8,), jnp.float32),
    mesh=_MESH,
    compiler_params=_CPARAMS,
    scratch_types=(
        pltpu.VMEM((6 * B1,), jnp.float32),
        pltpu.VMEM((6 * B2,), jnp.float32),
        pltpu.VMEM((6 * B1,), jnp.float32),
        pltpu.VMEM((6 * B2,), jnp.float32),
        pltpu.VMEM((16,), jnp.float32),
    ),
)
def _kernel_c(h1_hbm, h2_hbm, out_hbm, acc1_v, acc2_v, st1_v, st2_v, out_v):
    wid = _wid()

    @pl.when(wid == 0)
    def _():
        _accum_rows(h1_hbm, st1_v, acc1_v, 6 * B1, NC)
        _accum_rows(h2_hbm, st2_v, acc2_v, 6 * B2, NC)

        losses = []
        for k in range(3):
            count = _hist_count(acc1_v, k * 2 * B1, B1)
            nk = _n_keep(count)
            _, s1, c1 = _scan_top(acc1_v, k * 2 * B1, (k * 2 + 1) * B1,
                                  B1, nk)
            r1 = nk - c1
            b2, s2, c2 = _scan_top(acc2_v, k * 2 * B2, (k * 2 + 1) * B2,
                                   B2, r1)
            r2 = r1 - c2
            cnt_b2 = _scalar_at(acc2_v, k * 2 * B2 + b2)
            sum_b2 = _scalar_at(acc2_v, (k * 2 + 1) * B2 + b2)
            part = jnp.where(r2 > 0.5, r2 * _sdiv(sum_b2, cnt_b2), 0.0)
            total = s1 + s2 + part
            mean = _sdiv(total, nk)
            if k == 0:
                losses.append(mean)
            else:
                losses.append(jnp.where(count < 0.5, 0.0, mean))

        loss_cls, loss_off, loss_lmk = losses
        loss_total = CLS_W * loss_cls + BBOX_W * loss_off + LMK_W * loss_lmk
        lane = lax.iota(jnp.int32, L)
        zeros = jnp.zeros((L,), jnp.float32)
        ov = jnp.where(lane == 0, loss_total, zeros)
        ov = ov + jnp.where(lane == 1, loss_cls, zeros)
        ov = ov + jnp.where(lane == 2, loss_off, zeros)
        ov = ov + jnp.where(lane == 3, loss_lmk, zeros)
        out_v[pl.ds(0, L)] = ov
        pltpu.sync_copy(out_v.at[pl.ds(0, 8)], out_hbm)


def kernel(pred, labels, offsets, landmarks):
    vz, vso, vsl = _tc_values(pred.T, offsets.T, landmarks.T)
    vc, vo, vl, h1 = _kernel_h1(labels, vz, vso, vsl)
    h2 = _kernel_h2(vc, vo, vl, h1)
    out = _kernel_c(h1, h2)
    return (out[0], out[1], out[2], out[3])


# speedup vs baseline: 12.2213x; 1.0048x over previous
import functools

import jax
import jax.numpy as jnp
import numpy as np
from jax import lax
from jax.experimental import pallas as pl
from jax.experimental.pallas import tpu as pltpu
from jax.experimental.pallas import tpu_sc as plsc

N = 1048576
NC = 2
NS = 16
NW = NC * NS
L = 16
RW = N // NW

RTC = 16384
CH = 4096
NCH = RW // CH

B1 = 512
B2 = 512

CLS_W = 1.0
BBOX_W = 0.5
LMK_W = 0.5

_G_COEF = (1.8498544538905285e-04, 2.8751506391739456e-04,
           -5.4268610571399910e-03, 8.3107776364009530e-05,
           1.2498464620813230e-01, -4.9999884358222030e-01,
           6.9314715967354310e-01)

_MESH = plsc.VectorSubcoreMesh(core_axis_name="c", subcore_axis_name="s")
_CPARAMS = pltpu.CompilerParams(needs_layout_passes=False)


def _tc_body(pred_ref, off_ref, lmk_ref, vz_ref, vo_ref, vl_ref):
    pt = pred_ref[...]
    ot = off_ref[...]
    lt = lmk_ref[...]
    do = pt[1:5, :] - ot
    dl = pt[5:15, :] - lt
    vz_ref[...] = pt[0, :]
    vo_ref[...] = jnp.sum(do * do, axis=0)
    vl_ref[...] = jnp.sum(dl * dl, axis=0)


_tc_values = pl.pallas_call(
    _tc_body,
    grid=(N // RTC,),
    in_specs=[
        pl.BlockSpec((15, RTC), lambda i: (0, i)),
        pl.BlockSpec((4, RTC), lambda i: (0, i)),
        pl.BlockSpec((10, RTC), lambda i: (0, i)),
    ],
    out_specs=[pl.BlockSpec((RTC,), lambda i: (i,))] * 3,
    out_shape=[jax.ShapeDtypeStruct((N,), jnp.float32)] * 3,
)


def _wid():
    return lax.axis_index("s") * NC + lax.axis_index("c")


def _g_poly(s):
    acc = jnp.full(s.shape, _G_COEF[0], jnp.float32)
    for c in _G_COEF[1:]:
        acc = acc * s + c
    return acc


def _zero_ref(ref, nwords):
    z = jnp.zeros((L,), jnp.float32)

    @pl.loop(0, nwords // L)
    def _(i):
        ref[pl.ds(i * L, L)] = z


def _lane_fold(src, src_base, dst, dst_base, nbins, lane):
    @pl.loop(0, nbins // L)
    def _(i):
        bins = i * L + lane
        acc = jnp.zeros((L,), jnp.float32)
        for l in range(L):
            acc = acc + plsc.load_gather(src, [src_base + bins * L + l])
        dst[pl.ds(dst_base + i * L, L)] = acc


def _accum_rows(src_hbm, stage, acc, nwords, nrows):
    _zero_ref(acc, nwords)

    @pl.loop(0, nrows)
    def _(t):
        pltpu.sync_copy(src_hbm.at[pl.ds(t * nwords, nwords)], stage)

        @pl.loop(0, nwords // L)
        def _(i):
            sl = pl.ds(i * L, L)
            acc[sl] = acc[sl] + stage[sl]


def _combine_per_sc(fold_v, shared, stage, acc, out_hbm, nwords):
    sid = lax.axis_index("s")
    cid = lax.axis_index("c")
    pltpu.sync_copy(fold_v, shared.at[sid])
    plsc.subcore_barrier()

    @pl.when(sid == 0)
    def _():
        _zero_ref(acc, nwords)

        @pl.loop(0, NS)
        def _(t):
            pltpu.sync_copy(shared.at[t], stage)

            @pl.loop(0, nwords // L)
            def _(i):
                sl = pl.ds(i * L, L)
                acc[sl] = acc[sl] + stage[sl]

        pltpu.sync_copy(acc.at[pl.ds(0, nwords)],
                        out_hbm.at[pl.ds(cid * nwords, nwords)])


def _scan_top(ref, cnt_base, sum_base, nbins, target):
    nb = nbins // L

    def body(j, carry):
        found, b_star, s_above, c_above, ccnt, csum = carry
        vb = nb - 1 - j
        vc = ref[pl.ds(cnt_base + vb * L, L)]
        vs = ref[pl.ds(sum_base + vb * L, L)]
        rc = lax.rev(vc, (0,))
        rs = lax.rev(vs, (0,))
        cum = jnp.cumsum(rc) + ccnt
        m = cum >= target
        p = jnp.sum(jnp.where(m, 1.0, 0.0))
        has = (p > 0.5).astype(jnp.int32)
        b_here = vb * L + lax.convert_element_type(p, jnp.int32) - 1
        c_here = ccnt + jnp.sum(jnp.where(m, 0.0, rc))
        s_here = csum + jnp.sum(jnp.where(m, 0.0, rs))
        take = has * (1 - found)
        b_star = jnp.where(take > 0, b_here, b_star)
        s_above = jnp.where(take > 0, s_here, s_above)
        c_above = jnp.where(take > 0, c_here, c_above)
        found = jnp.maximum(found, has)
        ccnt = ccnt + jnp.sum(vc)
        csum = csum + jnp.sum(vs)
        return (found, b_star, s_above, c_above, ccnt, csum)

    init = (jnp.int32(0), jnp.int32(0), jnp.float32(0.0), jnp.float32(0.0),
            jnp.float32(0.0), jnp.float32(0.0))
    _, b_star, s_above, c_above, _, _ = lax.fori_loop(0, nb, body, init)
    return b_star, s_above, c_above


def _hist_count(ref, cnt_base, nbins):
    acc = jnp.zeros((L,), jnp.float32)

    def body(i, acc):
        return acc + ref[pl.ds(cnt_base + i * L, L)]

    acc = lax.fori_loop(0, nbins // L, body, acc)
    return jnp.sum(acc)


def _n_keep(count_f):
    ci = lax.convert_element_type(count_f, jnp.int32)
    nk = (7 * ci) // 10
    return lax.convert_element_type(nk, jnp.float32)


def _sdiv(a, b):
    q = jnp.full((L,), a, jnp.float32) / jnp.full((L,), b, jnp.float32)
    lane = lax.iota(jnp.int32, L)
    return jnp.sum(jnp.where(lane == 0, q, jnp.zeros((L,), jnp.float32)))


def _scalar_at(ref, idx):
    v = plsc.load_gather(ref, [jnp.full((L,), idx, jnp.int32)])
    return jnp.sum(v) * (1.0 / L)


@functools.partial(
    pl.kernel,
    out_type=(
        jax.ShapeDtypeStruct((N,), jnp.float32),
        jax.ShapeDtypeStruct((N,), jnp.float32),
        jax.ShapeDtypeStruct((N,), jnp.float32),
        jax.ShapeDtypeStruct((NC * 6 * B1,), jnp.float32),
    ),
    mesh=_MESH,
    compiler_params=_CPARAMS,
    scratch_types=(
        (pltpu.VMEM((CH,), jnp.int32),) * 2,
        (pltpu.VMEM((CH,), jnp.float32),) * 2,
        (pltpu.VMEM((CH,), jnp.float32),) * 2,
        (pltpu.VMEM((CH,), jnp.float32),) * 2,
        (pltpu.VMEM((CH,), jnp.float32),) * 2,
        (pltpu.VMEM((CH,), jnp.float32),) * 2,
        (pltpu.VMEM((CH,), jnp.float32),) * 2,
        pltpu.VMEM((6 * B1 * L,), jnp.float32),
        pltpu.VMEM((6 * B1,), jnp.float32),
        pltpu.VMEM((6 * B1,), jnp.float32),
        pltpu.VMEM((6 * B1,), jnp.float32),
        pltpu.VMEM_SHARED((NS, 6 * B1), jnp.float32),
        (pltpu.SemaphoreType.DMA,) * 2,
        (pltpu.SemaphoreType.DMA,) * 2,
    ),
)
def _kernel_h1(lab_hbm, vz_hbm, vso_hbm, vsl_hbm,
               vc_hbm, vo_hbm, vl_hbm, h1_hbm,
               lab_b, z_b, so_b, sl_b, oc_b, oo_b, ol_b,
               h_v, fold_v, stage_v, acc_v, shared, semi, semo):
    wid = _wid()
    lane = lax.iota(jnp.int32, L)
    ones = jnp.ones((L,), jnp.float32)
    neg1 = jnp.full((L,), -1.0, jnp.float32)
    c22 = jnp.full((L,), 22, jnp.int32)

    in_pairs = ((lab_hbm, lab_b), (vz_hbm, z_b), (vso_hbm, so_b),
                (vsl_hbm, sl_b))

    def start_in(ci, b):
        row0 = wid * RW + ci * CH
        for hbm, buf in in_pairs:
            pltpu.async_copy(hbm.at[pl.ds(row0, CH)], buf[b], semi[b])

    def wait_in(b):
        for hbm, buf in in_pairs:
            pltpu.make_async_copy(hbm.at[pl.ds(0, CH)], buf[b],
                                  semi[b]).wait()

    def start_out(ci, b):
        row0 = wid * RW + ci * CH
        pltpu.async_copy(oc_b[b], vc_hbm.at[pl.ds(row0, CH)], semo[b])
        pltpu.async_copy(oo_b[b], vo_hbm.at[pl.ds(row0, CH)], semo[b])
        pltpu.async_copy(ol_b[b], vl_hbm.at[pl.ds(row0, CH)], semo[b])

    def wait_out(b):
        for buf, hbm in ((oc_b, vc_hbm), (oo_b, vo_hbm), (ol_b, vl_hbm)):
            pltpu.make_async_copy(buf[b], hbm.at[pl.ds(0, CH)],
                                  semo[b]).wait()

    _zero_ref(h_v, 6 * B1 * L)
    start_in(0, 0)

    @pl.loop(0, NCH // 2)
    def _(oc):
        for b in range(2):
            ci = oc * 2 + b
            wait_in(b)

            @pl.when(ci + 1 < NCH)
            def _():
                start_in(ci + 1, 1 - b)

            @pl.when(ci >= 2)
            def _():
                wait_out(b)

            @pl.loop(0, CH // L, unroll=4)
            def _(g):
                sl = pl.ds(g * L, L)
                lbl = lab_b[b][sl]
                z = z_b[b][sl]
                so = so_b[b][sl]
                sl10 = sl_b[b][sl]

                s = 1.0 / (1.0 + jnp.exp(-z))
                y = jnp.where(lbl == 1, 1.0, 0.0)
                per_cls = s * (1.0 - y) + _g_poly(s)
                keep = lbl >= 0
                per_off = so * 0.25
                offm = (lbl == 1) | (lbl == -1)
                per_lmk = sl10 * 0.1
                lmkm = lbl == -2

                oc_b[b][sl] = jnp.where(keep, per_cls, neg1)
                oo_b[b][sl] = jnp.where(offm, per_off, neg1)
                ol_b[b][sl] = jnp.where(lmkm, per_lmk, neg1)

                for k, (per, msk) in enumerate(
                        ((per_cls, keep), (per_off, offm),
                         (per_lmk, lmkm))):
                    bits = plsc.bitcast(per, jnp.int32)
                    bb = lax.shift_right_logical(bits, c22)
                    idx = (k * 2 * B1 + bb) * L + lane
                    plsc.addupdate_scatter(h_v, [idx], ones, mask=msk)
                    plsc.addupdate_scatter(h_v, [idx + B1 * L], per,
                                           mask=msk)

            start_out(ci, b)

    for b in range(2):
        wait_out(b)

    for k in range(6):
        _lane_fold(h_v, k * B1 * L, fold_v, k * B1, B1, lane)
    _combine_per_sc(fold_v, shared, stage_v, acc_v, h1_hbm, 6 * B1)


@functools.partial(
    pl.kernel,
    out_type=jax.ShapeDtypeStruct((NC * 6 * B2,), jnp.float32),
    mesh=_MESH,
    compiler_params=_CPARAMS,
    scratch_types=(
        pltpu.VMEM((6 * B1,), jnp.float32),
        pltpu.VMEM((6 * B1,), jnp.float32),
        (pltpu.VMEM((CH,), jnp.float32),) * 2,
        (pltpu.VMEM((CH,), jnp.float32),) * 2,
        (pltpu.VMEM((CH,), jnp.float32),) * 2,
        pltpu.VMEM((6 * B2 * L,), jnp.float32),
        pltpu.VMEM((6 * B2,), jnp.float32),
        pltpu.VMEM_SHARED((NS, 6 * B2), jnp.float32),
        (pltpu.SemaphoreType.DMA,) * 2,
    ),
)
def _kernel_h2(vc_hbm, vo_hbm, vl_hbm, h1_hbm, h2_hbm,
               acc1_v, st1_v, bc_b, bo_b, bl_b, h2_v, fold_v, shared, semi):
    wid = _wid()
    lane = lax.iota(jnp.int32, L)
    ones = jnp.ones((L,), jnp.float32)
    c22 = jnp.full((L,), 22, jnp.int32)
    c13 = jnp.full((L,), 13, jnp.int32)

    def start_in(ci, b):
        row0 = wid * RW + ci * CH
        for buf, hbm in ((bc_b, vc_hbm), (bo_b, vo_hbm), (bl_b, vl_hbm)):
            pltpu.async_copy(hbm.at[pl.ds(row0, CH)], buf[b], semi[b])

    def wait_in(b):
        for buf, hbm in ((bc_b, vc_hbm), (bo_b, vo_hbm), (bl_b, vl_hbm)):
            pltpu.make_async_copy(hbm.at[pl.ds(0, CH)], buf[b],
                                  semi[b]).wait()

    _accum_rows(h1_hbm, st1_v, acc1_v, 6 * B1, NC)

    b1s = []
    for k in range(3):
        count = _hist_count(acc1_v, k * 2 * B1, B1)
        nk = _n_keep(count)
        b1, _, _ = _scan_top(acc1_v, k * 2 * B1, (k * 2 + 1) * B1, B1, nk)
        b1s.append(jnp.full((L,), b1, jnp.int32))

    _zero_ref(h2_v, 6 * B2 * L)
    start_in(0, 0)

    @pl.loop(0, NCH // 2)
    def _(oc):
        for b in range(2):
            ci = oc * 2 + b
            wait_in(b)

            @pl.when(ci + 1 < NCH)
            def _():
                start_in(ci + 1, 1 - b)

            @pl.loop(0, CH // L, unroll=4)
            def _(g):
                sl = pl.ds(g * L, L)
                for k, bufs in enumerate((bc_b, bo_b, bl_b)):
                    v = bufs[b][sl]
                    bits = plsc.bitcast(v, jnp.int32)
                    lvl1 = lax.shift_right_logical(bits, c22)
                    m = lvl1 == b1s[k]
                    sub = jnp.bitwise_and(
                        lax.shift_right_logical(bits, c13), B2 - 1)
                    idx = (k * 2 * B2 + sub) * L + lane
                    plsc.addupdate_scatter(h2_v, [idx], ones, mask=m)
                    plsc.addupdate_scatter(h2_v, [idx + B2 * L], v, mask=m)

    for k in range(6):
        _lane_fold(h2_v, k * B2 * L, fold_v, k * B2, B2, lane)
    _combine_per_sc(fold_v, shared, fold_v, h2_v, h2_hbm, 6 * B2)


@functools.partial(
    pl.kernel,
    out_type=jax.ShapeDtypeStruct((8,), jnp.float32),
    mesh=_MESH,
    compiler_params=_CPARAMS,
    scratch_types=(
        pltpu.VMEM((6 * B1,), jnp.float32),
        pltpu.VMEM((6 * B2,), jnp.float32),
        pltpu.VMEM((6 * B1,), jnp.float32),
        pltpu.VMEM((6 * B2,), jnp.float32),
        pltpu.VMEM((16,), jnp.float32),
    ),
)
def _kernel_c(h1_hbm, h2_hbm, out_hbm, acc1_v, acc2_v, st1_v, st2_v, out_v):
    wid = _wid()

    @pl.when(wid == 0)
    def _():
        _accum_rows(h1_hbm, st1_v, acc1_v, 6 * B1, NC)
        _accum_rows(h2_hbm, st2_v, acc2_v, 6 * B2, NC)

        losses = []
        for k in range(3):
            count = _hist_count(acc1_v, k * 2 * B1, B1)
            nk = _n_keep(count)
            _, s1, c1 = _scan_top(acc1_v, k * 2 * B1, (k * 2 + 1) * B1,
                                  B1, nk)
            r1 = nk - c1
            b2, s2, c2 = _scan_top(acc2_v, k * 2 * B2, (k * 2 + 1) * B2,
                                   B2, r1)
            r2 = r1 - c2
            cnt_b2 = _scalar_at(acc2_v, k * 2 * B2 + b2)
            sum_b2 = _scalar_at(acc2_v, (k * 2 + 1) * B2 + b2)
            part = jnp.where(r2 > 0.5, r2 * _sdiv(sum_b2, cnt_b2), 0.0)
            total = s1 + s2 + part
            mean = _sdiv(total, nk)
            if k == 0:
                losses.append(mean)
            else:
                losses.append(jnp.where(count < 0.5, 0.0, mean))

        loss_cls, loss_off, loss_lmk = losses
        loss_total = CLS_W * loss_cls + BBOX_W * loss_off + LMK_W * loss_lmk
        lane = lax.iota(jnp.int32, L)
        zeros = jnp.zeros((L,), jnp.float32)
        ov = jnp.where(lane == 0, loss_total, zeros)
        ov = ov + jnp.where(lane == 1, loss_cls, zeros)
        ov = ov + jnp.where(lane == 2, loss_off, zeros)
        ov = ov + jnp.where(lane == 3, loss_lmk, zeros)
        out_v[pl.ds(0, L)] = ov
        pltpu.sync_copy(out_v.at[pl.ds(0, 8)], out_hbm)


def kernel(pred, labels, offsets, landmarks):
    vz, vso, vsl = _tc_values(pred.T, offsets.T, landmarks.T)
    vc, vo, vl, h1 = _kernel_h1(labels, vz, vso, vsl)
    h2 = _kernel_h2(vc, vo, vl, h1)
    out = _kernel_c(h1, h2)
    return (out[0], out[1], out[2], out[3])
